# baseline (speedup 1.0000x reference)
import functools

import jax
import jax.numpy as jnp
from jax import lax
from jax.experimental import pallas as pl
from jax.experimental.pallas import tpu as pltpu
from jax.experimental.pallas import tpu_sc as plsc

VOCAB = 68
D = 1024
L = 2048
B = 32
B_SC = 8
B_TC = B - B_SC
NC = 2
NS = 16
NW = NC * NS
CL = 32
NRANGE = L // CL // NW
DV = D // 16
VP = 128
TB = 2048


def _positional_encoding():
    pos = jnp.arange(L, dtype=jnp.float32)[:, None]
    i = jnp.arange(0, D, 2, dtype=jnp.float32)
    denom = jnp.power(10000.0, i / D)
    ang = pos / denom[None, :]
    return jnp.stack([jnp.sin(ang), jnp.cos(ang)], axis=2).reshape(L, D)


def _sc_call(x_flat, pe, table):
    mesh = plsc.VectorSubcoreMesh(core_axis_name="c", subcore_axis_name="s")

    @functools.partial(
        pl.kernel,
        mesh=mesh,
        out_type=jax.ShapeDtypeStruct((B_SC * L, D), jnp.float32),
        scratch_types=[
            pltpu.VMEM((B_SC, CL), jnp.int32),
            pltpu.VMEM((CL, D), jnp.float32),
            pltpu.VMEM((CL, D), jnp.float32),
            pltpu.VMEM((CL, D), jnp.float32),
            pltpu.SemaphoreType.DMA,
            pltpu.SemaphoreType.DMA,
            pltpu.SemaphoreType.DMA,
            pltpu.SemaphoreType.DMA,
            pltpu.SemaphoreType.DMA,
        ],
    )
    def emb_kernel(x_hbm, pe_hbm, table_hbm, out_hbm,
                   idx_v, rows_a, rows_b, pe_v,
                   gsem_a, gsem_b, wsem_a, wsem_b, isem):
        cid = lax.axis_index("c")
        sid = lax.axis_index("s")
        wid = sid * NC + cid

        def gather(b, rows, gsem):
            pltpu.async_copy(table_hbm.at[idx_v.at[b]], rows, gsem)

        def gather_wait(rows, gsem):
            pltpu.make_async_copy(table_hbm.at[idx_v.at[0]], rows, gsem).wait()

        def add_pe(rows):
            def row_body(r, c2):
                for k in range(DV):
                    v = pe_v[r, pl.ds(16 * k, 16)]
                    plsc.addupdate(rows.at[r, pl.ds(16 * k, 16)], v)
                return c2
            lax.fori_loop(0, CL, row_body, 0)

        def writeout(b, l0, rows, wsem):
            pltpu.async_copy(rows, out_hbm.at[pl.ds(b * L + l0, CL)], wsem)

        def writeout_wait(rows, wsem):
            pltpu.make_async_copy(rows, out_hbm.at[pl.ds(0, CL)], wsem).wait()

        for rng in range(NRANGE):
            l0 = (rng * NW + wid) * CL
            for b in range(B_SC):
                pltpu.async_copy(x_hbm.at[pl.ds((B_TC + b) * L + l0, CL)],
                                 idx_v.at[b], isem)
            pltpu.sync_copy(pe_hbm.at[pl.ds(l0, CL)], pe_v)
            for b in range(B_SC):
                pltpu.make_async_copy(x_hbm.at[pl.ds((B_TC + b) * L + l0, CL)],
                                      idx_v.at[b], isem).wait()
            gather(0, rows_a, gsem_a)

            def pair_body(j, carry):
                b0 = 2 * j
                @pl.when(j > 0)
                def _wb():
                    writeout_wait(rows_b, wsem_b)
                gather(b0 + 1, rows_b, gsem_b)
                gather_wait(rows_a, gsem_a)
                add_pe(rows_a)
                writeout(b0, l0, rows_a, wsem_a)
                writeout_wait(rows_a, wsem_a)
                @pl.when(j < B_SC // 2 - 1)
                def _g():
                    gather(b0 + 2, rows_a, gsem_a)
                gather_wait(rows_b, gsem_b)
                add_pe(rows_b)
                writeout(b0 + 1, l0, rows_b, wsem_b)
                return carry

            lax.fori_loop(0, B_SC // 2, pair_body, 0)
            writeout_wait(rows_b, wsem_b)

    return emb_kernel(x_flat, pe, table)


def _tc_body(x_ref, tab_ref, pe_ref, o_ref):
    xv = x_ref[...]
    iot = lax.broadcasted_iota(jnp.int32, (TB, VP), 1)
    onehot = (iot == xv).astype(jnp.float32)
    emb = jnp.dot(onehot, tab_ref[...], preferred_element_type=jnp.float32)
    o_ref[...] = emb + pe_ref[...]


def _tc_call(x_col, tab_pad, pe):
    return pl.pallas_call(
        _tc_body,
        grid=(B_TC * L // TB,),
        in_specs=[
            pl.BlockSpec((TB, 1), lambda j: (j, 0)),
            pl.BlockSpec((VP, D), lambda j: (0, 0)),
            pl.BlockSpec((TB, D), lambda j: (0, 0)),
        ],
        out_specs=pl.BlockSpec((TB, D), lambda j: (j, 0)),
        out_shape=jax.ShapeDtypeStruct((B_TC * L, D), jnp.float32),
    )(x_col, tab_pad, pe)


def kernel(x, table):
    pe = _positional_encoding()
    x_flat = x.reshape(B * L).astype(jnp.int32)
    x_col_tc = x_flat[:B_TC * L].reshape(B_TC * L, 1)
    tab_pad = jnp.zeros((VP, D), jnp.float32).at[:VOCAB].set(table)

    out_sc = _sc_call(x_flat, pe, table)
    out_tc = _tc_call(x_col_tc, tab_pad, pe)
    out = jnp.concatenate([out_tc, out_sc], axis=0)
    return out.reshape(B, L, D)


# speedup vs baseline: 1.2084x; 1.2084x over previous
import functools

import jax
import jax.numpy as jnp
from jax import lax
from jax.experimental import pallas as pl
from jax.experimental.pallas import tpu as pltpu
from jax.experimental.pallas import tpu_sc as plsc

VOCAB = 68
D = 1024
L = 2048
B = 32
B_SC = 2
B_TC = B - B_SC
NC = 2
NS = 16
NW = NC * NS
CL = 32
NRANGE = L // CL // NW
DV = D // 16
VP = 128
TB = 2048


def _positional_encoding():
    pos = jnp.arange(L, dtype=jnp.float32)[:, None]
    i = jnp.arange(0, D, 2, dtype=jnp.float32)
    denom = jnp.power(10000.0, i / D)
    ang = pos / denom[None, :]
    return jnp.stack([jnp.sin(ang), jnp.cos(ang)], axis=2).reshape(L, D)


def _sc_call(x_flat, pe, table):
    mesh = plsc.VectorSubcoreMesh(core_axis_name="c", subcore_axis_name="s")

    @functools.partial(
        pl.kernel,
        mesh=mesh,
        out_type=jax.ShapeDtypeStruct((B_SC * L, D), jnp.float32),
        scratch_types=[
            pltpu.VMEM((B_SC, CL), jnp.int32),
            pltpu.VMEM((CL, D), jnp.float32),
            pltpu.VMEM((CL, D), jnp.float32),
            pltpu.VMEM((CL, D), jnp.float32),
            pltpu.SemaphoreType.DMA,
            pltpu.SemaphoreType.DMA,
            pltpu.SemaphoreType.DMA,
            pltpu.SemaphoreType.DMA,
            pltpu.SemaphoreType.DMA,
        ],
    )
    def emb_kernel(x_hbm, pe_hbm, table_hbm, out_hbm,
                   idx_v, rows_a, rows_b, pe_v,
                   gsem_a, gsem_b, wsem_a, wsem_b, isem):
        cid = lax.axis_index("c")
        sid = lax.axis_index("s")
        wid = sid * NC + cid

        def gather(b, rows, gsem):
            pltpu.async_copy(table_hbm.at[idx_v.at[b]], rows, gsem)

        def gather_wait(rows, gsem):
            pltpu.make_async_copy(table_hbm.at[idx_v.at[0]], rows, gsem).wait()

        def add_pe(rows):
            def row_body(r, c2):
                for k in range(DV):
                    v = pe_v[r, pl.ds(16 * k, 16)]
                    plsc.addupdate(rows.at[r, pl.ds(16 * k, 16)], v)
                return c2
            lax.fori_loop(0, CL, row_body, 0)

        def writeout(b, l0, rows, wsem):
            pltpu.async_copy(rows, out_hbm.at[pl.ds(b * L + l0, CL)], wsem)

        def writeout_wait(rows, wsem):
            pltpu.make_async_copy(rows, out_hbm.at[pl.ds(0, CL)], wsem).wait()

        for rng in range(NRANGE):
            l0 = (rng * NW + wid) * CL
            for b in range(B_SC):
                pltpu.async_copy(x_hbm.at[pl.ds((B_TC + b) * L + l0, CL)],
                                 idx_v.at[b], isem)
            pltpu.sync_copy(pe_hbm.at[pl.ds(l0, CL)], pe_v)
            for b in range(B_SC):
                pltpu.make_async_copy(x_hbm.at[pl.ds((B_TC + b) * L + l0, CL)],
                                      idx_v.at[b], isem).wait()
            gather(0, rows_a, gsem_a)

            def pair_body(j, carry):
                b0 = 2 * j
                @pl.when(j > 0)
                def _wb():
                    writeout_wait(rows_b, wsem_b)
                gather(b0 + 1, rows_b, gsem_b)
                gather_wait(rows_a, gsem_a)
                add_pe(rows_a)
                writeout(b0, l0, rows_a, wsem_a)
                writeout_wait(rows_a, wsem_a)
                @pl.when(j < B_SC // 2 - 1)
                def _g():
                    gather(b0 + 2, rows_a, gsem_a)
                gather_wait(rows_b, gsem_b)
                add_pe(rows_b)
                writeout(b0 + 1, l0, rows_b, wsem_b)
                return carry

            lax.fori_loop(0, B_SC // 2, pair_body, 0)
            writeout_wait(rows_b, wsem_b)

    return emb_kernel(x_flat, pe, table)


def _tc_body(x_ref, tab_ref, pe_ref, o_ref):
    xv = x_ref[...]
    iot = lax.broadcasted_iota(jnp.int32, (TB, VP), 1)
    onehot = (iot == xv).astype(jnp.float32)
    emb = jnp.dot(onehot, tab_ref[...], preferred_element_type=jnp.float32)
    o_ref[...] = emb + pe_ref[...]


def _tc_call(x_col, tab_pad, pe):
    return pl.pallas_call(
        _tc_body,
        grid=(B_TC * L // TB,),
        in_specs=[
            pl.BlockSpec((TB, 1), lambda j: (j, 0)),
            pl.BlockSpec((VP, D), lambda j: (0, 0)),
            pl.BlockSpec((TB, D), lambda j: (0, 0)),
        ],
        out_specs=pl.BlockSpec((TB, D), lambda j: (j, 0)),
        out_shape=jax.ShapeDtypeStruct((B_TC * L, D), jnp.float32),
    )(x_col, tab_pad, pe)


def kernel(x, table):
    pe = _positional_encoding()
    x_flat = x.reshape(B * L).astype(jnp.int32)
    x_col_tc = x_flat[:B_TC * L].reshape(B_TC * L, 1)
    tab_pad = jnp.zeros((VP, D), jnp.float32).at[:VOCAB].set(table)

    out_sc = _sc_call(x_flat, pe, table)
    out_tc = _tc_call(x_col_tc, tab_pad, pe)
    out = jnp.concatenate([out_tc, out_sc], axis=0)
    return out.reshape(B, L, D)
